# baseline (speedup 1.0000x reference)
import functools

import jax
import jax.numpy as jnp
from jax.experimental import pallas as pl
from jax.experimental.pallas import tpu as pltpu


def _merge_body(seq_ref, kb, nk, vb, nv, ok, ov, *, bc, t_rows):
    i = pl.program_id(0)
    seq = seq_ref[0]
    row = i * bc + jax.lax.broadcasted_iota(jnp.int32, kb.shape, 0)
    in_new = (row >= seq) & (row < seq + t_rows)
    ok[...] = jnp.where(in_new, nk[...], kb[...])
    ov[...] = jnp.where(in_new, nv[...], vb[...])


def kernel(keys_buffer, values_buffer, new_keys, new_values, sequence_length):
    C, G, D = keys_buffer.shape
    T = new_keys.shape[0]
    GD = G * D
    BC = 256
    nb = C // BC
    tnb = T // BC

    kb2 = keys_buffer.reshape(C, GD)
    vb2 = values_buffer.reshape(C, GD)
    nk2 = new_keys.reshape(T, GD)
    nv2 = new_values.reshape(T, GD)
    seq = jnp.asarray(sequence_length, jnp.int32).reshape(1)

    def buf_map(i, s):
        seq_b = s[0] // BC
        hi_b = seq_b + tnb
        interior = jnp.maximum(seq_b - 1, 0)
        return (jnp.where((i < seq_b) | (i >= hi_b), i, interior), 0)

    def new_map(i, s):
        seq_b = s[0] // BC
        return (jnp.clip(i - seq_b, 0, tnb - 1), 0)

    grid_spec = pltpu.PrefetchScalarGridSpec(
        num_scalar_prefetch=1,
        grid=(nb,),
        in_specs=[
            pl.BlockSpec((BC, GD), buf_map),
            pl.BlockSpec((BC, GD), new_map),
            pl.BlockSpec((BC, GD), buf_map),
            pl.BlockSpec((BC, GD), new_map),
        ],
        out_specs=[
            pl.BlockSpec((BC, GD), lambda i, s: (i, 0)),
            pl.BlockSpec((BC, GD), lambda i, s: (i, 0)),
        ],
    )

    out_k, out_v = pl.pallas_call(
        functools.partial(_merge_body, bc=BC, t_rows=T),
        grid_spec=grid_spec,
        out_shape=[
            jax.ShapeDtypeStruct((C, GD), keys_buffer.dtype),
            jax.ShapeDtypeStruct((C, GD), values_buffer.dtype),
        ],
        compiler_params=pltpu.CompilerParams(
            dimension_semantics=("arbitrary",),
        ),
    )(seq, kb2, nk2, vb2, nv2)

    new_seq = jnp.asarray(sequence_length + T, dtype=jnp.int32)
    return (new_seq, out_k.reshape(C, G, D), out_v.reshape(C, G, D))
